# baseline (speedup 1.0000x reference)
import functools

import jax
import jax.numpy as jnp
from jax import lax
from jax.experimental import pallas as pl
from jax.experimental.pallas import tpu as pltpu

N = 50000
E = 800000
H = 128
EH = 64
L = 4

_INTERPRET = False


def _encoder_body(x_ref, wn_ref, bn_ref, h_ref):
    h_ref[...] = jax.nn.gelu(
        jnp.dot(x_ref[...], wn_ref[...], preferred_element_type=jnp.float32)
        + bn_ref[...]
    )


def _node_encoder(x, Wn, bn):
    n, nd = x.shape
    bs = 2000
    grid = n // bs
    return pl.pallas_call(
        _encoder_body,
        grid=(grid,),
        in_specs=[
            pl.BlockSpec((bs, nd), lambda i: (i, 0)),
            pl.BlockSpec((nd, H), lambda i: (0, 0)),
            pl.BlockSpec((1, H), lambda i: (0, 0)),
        ],
        out_specs=pl.BlockSpec((bs, H), lambda i: (i, 0)),
        out_shape=jax.ShapeDtypeStruct((n, H), jnp.float32),
        interpret=_INTERPRET,
    )(x, Wn, bn.reshape(1, H))


def _edge_pre_body(ea_ref, we_ref, be_ref, wme_ref, *out_refs):
    e = jax.nn.gelu(
        jnp.dot(ea_ref[...], we_ref[...], preferred_element_type=jnp.float32)
        + be_ref[...]
    )
    for l, o_ref in enumerate(out_refs):
        o_ref[...] = jnp.dot(e, wme_ref[l], preferred_element_type=jnp.float32)


def _edge_pre(edge_attr, We, be, Wm):
    e_rows, ed = edge_attr.shape
    bs = 8000
    grid = e_rows // bs
    wme = Wm[:, 2 * H :, :]
    return pl.pallas_call(
        _edge_pre_body,
        grid=(grid,),
        in_specs=[
            pl.BlockSpec((bs, ed), lambda i: (i, 0)),
            pl.BlockSpec((ed, EH), lambda i: (0, 0)),
            pl.BlockSpec((1, EH), lambda i: (0, 0)),
            pl.BlockSpec((L, EH, H), lambda i: (0, 0, 0)),
        ],
        out_specs=[pl.BlockSpec((bs, H), lambda i: (i, 0)) for _ in range(L)],
        out_shape=[jax.ShapeDtypeStruct((e_rows, H), jnp.float32) for _ in range(L)],
        interpret=_INTERPRET,
    )(edge_attr, We, be.reshape(1, EH), wme)


def _proj_body(h_ref, ws_ref, wd_ref, bm_ref, hs_ref, hd_ref):
    h = h_ref[...]
    hs_ref[...] = jnp.dot(h, ws_ref[...], preferred_element_type=jnp.float32)
    hd_ref[...] = (
        jnp.dot(h, wd_ref[...], preferred_element_type=jnp.float32) + bm_ref[...]
    )


def _node_proj(h, Wm_l, bm_l):
    n = h.shape[0]
    bs = 2000
    grid = n // bs
    ws = Wm_l[:H, :]
    wd = Wm_l[H : 2 * H, :]
    return pl.pallas_call(
        _proj_body,
        grid=(grid,),
        in_specs=[
            pl.BlockSpec((bs, H), lambda i: (i, 0)),
            pl.BlockSpec((H, H), lambda i: (0, 0)),
            pl.BlockSpec((H, H), lambda i: (0, 0)),
            pl.BlockSpec((1, H), lambda i: (0, 0)),
        ],
        out_specs=[
            pl.BlockSpec((bs, H), lambda i: (i, 0)),
            pl.BlockSpec((bs, H), lambda i: (i, 0)),
        ],
        out_shape=[
            jax.ShapeDtypeStruct((n, H), jnp.float32),
            jax.ShapeDtypeStruct((n, H), jnp.float32),
        ],
        interpret=_INTERPRET,
    )(h, ws, wd, bm_l.reshape(1, H))


def _msg_body(a_ref, b_ref, c_ref, o_ref):
    o_ref[...] = jax.nn.gelu(a_ref[...] + b_ref[...] + c_ref[...])


def _edge_msg(hs_g, hd_g, ep):
    e_rows = hs_g.shape[0]
    bs = 8000
    grid = e_rows // bs
    spec = pl.BlockSpec((bs, H), lambda i: (i, 0))
    return pl.pallas_call(
        _msg_body,
        grid=(grid,),
        in_specs=[spec, spec, spec],
        out_specs=spec,
        out_shape=jax.ShapeDtypeStruct((e_rows, H), jnp.float32),
        interpret=_INTERPRET,
    )(hs_g, hd_g, ep)


def _update_body(h_ref, agg_ref, wa_ref, wb_ref, bu_ref, o_ref):
    h = h_ref[...]
    o_ref[...] = (
        jax.nn.gelu(
            jnp.dot(h, wa_ref[...], preferred_element_type=jnp.float32)
            + jnp.dot(agg_ref[...], wb_ref[...], preferred_element_type=jnp.float32)
            + bu_ref[...]
        )
        + h
    )


def _node_update(h, agg, Wu_l, bu_l):
    n = h.shape[0]
    bs = 2000
    grid = n // bs
    wa = Wu_l[:H, :]
    wb = Wu_l[H:, :]
    return pl.pallas_call(
        _update_body,
        grid=(grid,),
        in_specs=[
            pl.BlockSpec((bs, H), lambda i: (i, 0)),
            pl.BlockSpec((bs, H), lambda i: (i, 0)),
            pl.BlockSpec((H, H), lambda i: (0, 0)),
            pl.BlockSpec((H, H), lambda i: (0, 0)),
            pl.BlockSpec((1, H), lambda i: (0, 0)),
        ],
        out_specs=pl.BlockSpec((bs, H), lambda i: (i, 0)),
        out_shape=jax.ShapeDtypeStruct((n, H), jnp.float32),
        interpret=_INTERPRET,
    )(h, agg, wa, wb, bu_l.reshape(1, H))


def _readout_body(h_ref, wp_ref, bp_ref, wv1_ref, bv1_ref, wv2_ref, bv2_ref,
                  logits_ref, value_ref, sum_ref, max_ref):
    i = pl.program_id(0)
    nsteps = pl.num_programs(0)
    h = h_ref[...]

    @pl.when(i == 0)
    def _init():
        sum_ref[...] = jnp.zeros_like(sum_ref)
        max_ref[...] = jnp.full_like(max_ref, -jnp.inf)

    sum_ref[...] += jnp.sum(h, axis=0, keepdims=True)
    max_ref[...] = jnp.maximum(max_ref[...], jnp.max(h, axis=0, keepdims=True))

    @pl.when(i == nsteps - 1)
    def _fin():
        g = jnp.concatenate([sum_ref[...] / N, max_ref[...]], axis=-1)
        logits_ref[...] = (
            jnp.dot(g, wp_ref[...], preferred_element_type=jnp.float32) + bp_ref[...]
        )
        v = jax.nn.gelu(
            jnp.dot(g, wv1_ref[...], preferred_element_type=jnp.float32)
            + bv1_ref[...]
        )
        value_ref[...] = (
            jnp.dot(v, wv2_ref[...], preferred_element_type=jnp.float32) + bv2_ref[...]
        )


def _readout(h, Wp, bp, Wv1, bv1, Wv2, bv2):
    n = h.shape[0]
    bs = 2000
    grid = n // bs
    a = Wp.shape[1]
    return pl.pallas_call(
        _readout_body,
        grid=(grid,),
        in_specs=[
            pl.BlockSpec((bs, H), lambda i: (i, 0)),
            pl.BlockSpec((2 * H, a), lambda i: (0, 0)),
            pl.BlockSpec((1, a), lambda i: (0, 0)),
            pl.BlockSpec((2 * H, H), lambda i: (0, 0)),
            pl.BlockSpec((1, H), lambda i: (0, 0)),
            pl.BlockSpec((H, 1), lambda i: (0, 0)),
            pl.BlockSpec((1, 1), lambda i: (0, 0)),
        ],
        out_specs=[
            pl.BlockSpec((1, a), lambda i: (0, 0)),
            pl.BlockSpec((1, 1), lambda i: (0, 0)),
        ],
        out_shape=[
            jax.ShapeDtypeStruct((1, a), jnp.float32),
            jax.ShapeDtypeStruct((1, 1), jnp.float32),
        ],
        scratch_shapes=[
            pltpu.VMEM((1, H), jnp.float32),
            pltpu.VMEM((1, H), jnp.float32),
        ],
        interpret=_INTERPRET,
    )(h, Wp, bp.reshape(1, a), Wv1, bv1.reshape(1, H), Wv2, bv2.reshape(1, 1))


def _edge_stage(hs, hd, ep_l, src, dst):
    hs_g = jnp.take(hs, src, axis=0)
    hd_g = jnp.take(hd, dst, axis=0)
    m = _edge_msg(hs_g, hd_g, ep_l)
    return jnp.zeros_like(hs).at[dst].add(m)


def kernel(x, edge_index, edge_attr, Wn, bn, We, be, Wm, bm, Wu, bu, Wp, bp,
           Wv1, bv1, Wv2, bv2):
    src = edge_index[0]
    dst = edge_index[1]
    h = _node_encoder(x, Wn, bn)
    eps = _edge_pre(edge_attr, We, be, Wm)
    for l in range(L):
        hs, hd = _node_proj(h, Wm[l], bm[l])
        agg = _edge_stage(hs, hd, eps[l], src, dst)
        h = _node_update(h, agg, Wu[l], bu[l])
    logits, value = _readout(h, Wp, bp, Wv1, bv1, Wv2, bv2)
    return (logits, value)
